# baseline (speedup 1.0000x reference)
import jax
import jax.numpy as jnp
from jax import lax
from jax.experimental import pallas as pl
from jax.experimental.pallas import tpu as pltpu
from jax.experimental.pallas import tpu_sc as plsc

NUM_CORES = 2
NUM_SUBCORES = 16
NUM_WORKERS = NUM_CORES * NUM_SUBCORES
CHUNK_B = 32
SEQ_PAD = 24


def _gather_kernel(table_hbm, ids_hbm, out_hbm, idx_v, rows_v, sem):
    n_batch, seq, _ = out_hbm.shape
    b_per_w = n_batch // NUM_WORKERS
    wid = lax.axis_index("s") * NUM_CORES + lax.axis_index("c")
    b0w = wid * b_per_w

    zeros = jnp.zeros((16,), jnp.int32)

    @pl.loop(0, CHUNK_B * SEQ_PAD, step=16)
    def _(j):
        idx_v[pl.ds(j, 16)] = zeros

    @pl.loop(0, b_per_w, step=CHUNK_B)
    def _(bo):
        b0 = b0w + bo
        id_handles = [
            pltpu.async_copy(
                ids_hbm.at[b0 + i], idx_v.at[pl.ds(i * SEQ_PAD, seq)], sem
            )
            for i in range(CHUNK_B)
        ]
        for h in id_handles:
            h.wait()
        pltpu.async_copy(table_hbm.at[idx_v], rows_v, sem).wait()
        out_handles = [
            pltpu.async_copy(
                rows_v.at[pl.ds(i * SEQ_PAD, seq)], out_hbm.at[b0 + i], sem
            )
            for i in range(CHUNK_B)
        ]
        for h in out_handles:
            h.wait()


def kernel(token_ids, embedding_table):
    batch, seq = token_ids.shape
    dim = embedding_table.shape[1]
    ids = token_ids.astype(jnp.int32)

    mesh = plsc.VectorSubcoreMesh(core_axis_name="c", subcore_axis_name="s")
    k = pl.kernel(
        _gather_kernel,
        mesh=mesh,
        out_type=jax.ShapeDtypeStruct((batch, seq, dim), embedding_table.dtype),
        scratch_types=[
            pltpu.VMEM((CHUNK_B * SEQ_PAD,), jnp.int32),
            pltpu.VMEM((CHUNK_B * SEQ_PAD, dim), jnp.float32),
            pltpu.SemaphoreType.DMA,
        ],
        compiler_params=pltpu.CompilerParams(use_tc_tiling_on_sc=False),
    )
    return k(embedding_table, ids)


# speedup vs baseline: 2.4799x; 2.4799x over previous
import jax
import jax.numpy as jnp
from jax import lax
from jax.experimental import pallas as pl
from jax.experimental.pallas import tpu as pltpu
from jax.experimental.pallas import tpu_sc as plsc

NUM_CORES = 2
NUM_SUBCORES = 16
NUM_WORKERS = NUM_CORES * NUM_SUBCORES
CHUNK_B = 32


def _gather_kernel(table_hbm, ids_hbm, out_hbm, idx2_v, idx_v, rows_v, sem):
    n_batch, seq, _ = out_hbm.shape
    b_per_w = n_batch // NUM_WORKERS
    wid = lax.axis_index("s") * NUM_CORES + lax.axis_index("c")
    b0w = wid * b_per_w
    n_ids = CHUNK_B * seq
    lane = lax.iota(jnp.int32, 16)

    @pl.loop(0, b_per_w, step=CHUNK_B)
    def _(bo):
        b0 = b0w + bo
        pltpu.sync_copy(ids_hbm.at[pl.ds(b0, CHUNK_B)], idx2_v)
        for j in range(n_ids // 16):
            flat = lane + (16 * j)
            vals = plsc.load_gather(idx2_v, [flat // seq, flat % seq])
            idx_v[pl.ds(16 * j, 16)] = vals
        pltpu.async_copy(table_hbm.at[idx_v], rows_v, sem).wait()
        out_handles = [
            pltpu.async_copy(
                rows_v.at[pl.ds(i * seq, seq)], out_hbm.at[b0 + i], sem
            )
            for i in range(CHUNK_B)
        ]
        for h in out_handles:
            h.wait()


def kernel(token_ids, embedding_table):
    batch, seq = token_ids.shape
    dim = embedding_table.shape[1]
    ids = token_ids.astype(jnp.int32)

    mesh = plsc.VectorSubcoreMesh(core_axis_name="c", subcore_axis_name="s")
    k = pl.kernel(
        _gather_kernel,
        mesh=mesh,
        out_type=jax.ShapeDtypeStruct((batch, seq, dim), embedding_table.dtype),
        scratch_types=[
            pltpu.VMEM((CHUNK_B, seq), jnp.int32),
            pltpu.VMEM((CHUNK_B * seq,), jnp.int32),
            pltpu.VMEM((CHUNK_B * seq, dim), jnp.float32),
            pltpu.SemaphoreType.DMA,
        ],
        compiler_params=pltpu.CompilerParams(
            use_tc_tiling_on_sc=False, needs_layout_passes=False
        ),
    )
    return k(embedding_table, ids)
